# baseline (speedup 1.0000x reference)
import functools

import jax
import jax.numpy as jnp
import ml_dtypes
import numpy as np
from jax.experimental import pallas as pl
from jax.experimental.pallas import tpu as pltpu

_A = 0.999
_OMA = 1.0 - _A
_EPS = 1e-05
_C = 128
_PAD = 8


@functools.lru_cache(maxsize=None)
def _chunk_mats(C: int):
    j = np.arange(C, dtype=np.float64)[:, None]
    k = np.arange(C, dtype=np.float64)[None, :]
    L = np.where(k < j, _A ** np.maximum(j - 1 - k, 0.0), 0.0)
    Tm = np.zeros((C, C + _PAD), np.float32)
    Tv = np.zeros((C, C + _PAD), np.float32)
    pow_j = (_A ** np.arange(C, dtype=np.float64)).astype(np.float32)
    Tm[:, 0] = pow_j
    Tm[:, 1] = pow_j
    Tm[:, _PAD:] = _OMA * L
    Tv[:, _PAD:] = L
    POW = np.repeat(pow_j[:, None], 128, axis=1)
    Tm_hi = Tm.astype(ml_dtypes.bfloat16).astype(np.float32)
    Tm_lo = Tm - Tm_hi
    return (jnp.asarray(Tm_hi), jnp.asarray(Tm_lo), jnp.asarray(Tv),
            jnp.asarray(POW))


def _body(x_ref, m0_ref, v0_ref, tmh_ref, tml_ref, tv_ref, pow_ref,
          out_ref, mout_ref, vout_ref, rm_ref, rv_ref, vc_ref):
    b = pl.program_id(1)

    @pl.when(b == 0)
    def _init():
        rm_ref[0:_PAD, :] = jnp.zeros_like(rm_ref[0:_PAD, :])
        rv_ref[0:_PAD, :] = jnp.zeros_like(rv_ref[0:_PAD, :])
        m0 = m0_ref[...]
        hi = m0.astype(jnp.bfloat16).astype(jnp.float32)
        rm_ref[0:1, :] = hi
        rm_ref[1:2, :] = m0 - hi
        vc_ref[...] = v0_ref[...]

    C = _C
    Fb = out_ref.shape[1]
    xb = x_ref[...]
    rm_ref[_PAD:, :] = xb
    rm = rm_ref[...]
    m = jax.lax.dot_general(
        tmh_ref[...], rm, (((1,), (0,)), ((), ())),
        precision=jax.lax.Precision.DEFAULT,
        preferred_element_type=jnp.float32) + jax.lax.dot_general(
        tml_ref[...], rm, (((1,), (0,)), ((), ())),
        precision=jax.lax.Precision.DEFAULT,
        preferred_element_type=jnp.float32)
    d = xb - m
    bb = (_A * _OMA) * (d * d)
    rv_ref[_PAD:, :] = bb
    powf = pltpu.repeat(pow_ref[...], Fb // 128, axis=1)
    vcarry = jnp.broadcast_to(vc_ref[...], (C, Fb))
    v = jax.lax.dot_general(
        tv_ref[...], rv_ref[...], (((1,), (0,)), ((), ())),
        precision=jax.lax.Precision.DEFAULT,
        preferred_element_type=jnp.float32) + powf * vcarry
    out_ref[...] = d * jax.lax.rsqrt(v + _EPS)

    d_last = d[C - 1:C, :]
    m_carry = m[C - 1:C, :] + _OMA * d_last
    v_carry = _A * v[C - 1:C, :] + (_A * _OMA) * (d_last * d_last)
    hi = m_carry.astype(jnp.bfloat16).astype(jnp.float32)
    rm_ref[0:1, :] = hi
    rm_ref[1:2, :] = m_carry - hi
    vc_ref[...] = v_carry
    mout_ref[...] = m_carry
    vout_ref[...] = v_carry


def kernel(x, mstream, varstream):
    B, F = x.shape
    C = _C
    Fb = F // 2 if F % 256 == 0 and F >= 512 else F
    nb = B // C
    nf = F // Fb
    Tm_hi, Tm_lo, Tv, POW = _chunk_mats(C)
    m2 = mstream.reshape(1, F)
    v2 = varstream.reshape(1, F)

    out, mfin, vfin = pl.pallas_call(
        _body,
        grid=(nf, nb),
        in_specs=[
            pl.BlockSpec((C, Fb), lambda f, b: (b, f)),
            pl.BlockSpec((1, Fb), lambda f, b: (0, f)),
            pl.BlockSpec((1, Fb), lambda f, b: (0, f)),
            pl.BlockSpec((C, C + _PAD), lambda f, b: (0, 0)),
            pl.BlockSpec((C, C + _PAD), lambda f, b: (0, 0)),
            pl.BlockSpec((C, C + _PAD), lambda f, b: (0, 0)),
            pl.BlockSpec((C, 128), lambda f, b: (0, 0)),
        ],
        out_specs=[
            pl.BlockSpec((C, Fb), lambda f, b: (b, f)),
            pl.BlockSpec((1, Fb), lambda f, b: (0, f)),
            pl.BlockSpec((1, Fb), lambda f, b: (0, f)),
        ],
        out_shape=[
            jax.ShapeDtypeStruct((B, F), jnp.float32),
            jax.ShapeDtypeStruct((1, F), jnp.float32),
            jax.ShapeDtypeStruct((1, F), jnp.float32),
        ],
        scratch_shapes=[
            pltpu.VMEM((C + _PAD, Fb), jnp.float32),
            pltpu.VMEM((C + _PAD, Fb), jnp.float32),
            pltpu.VMEM((1, Fb), jnp.float32),
        ],
        compiler_params=pltpu.CompilerParams(
            dimension_semantics=("parallel", "arbitrary")),
    )(x, m2, v2, Tm_hi, Tm_lo, Tv, POW)
    return out, mfin.reshape(F), vfin.reshape(F)


# speedup vs baseline: 1.3860x; 1.3860x over previous
import functools

import jax
import jax.numpy as jnp
import numpy as np
from jax.experimental import pallas as pl
from jax.experimental.pallas import tpu as pltpu

_A = 0.999
_OMA = 1.0 - _A
_EPS = 1e-05
_C = 256
_PAD = 8


@functools.lru_cache(maxsize=None)
def _chunk_mats(C: int):
    j = np.arange(C, dtype=np.float64)[:, None]
    k = np.arange(C, dtype=np.float64)[None, :]
    L = np.where(k < j, _A ** np.maximum(j - 1 - k, 0.0), 0.0)
    Tm = np.zeros((C, C + _PAD), np.float32)
    Tv = np.zeros((C, C + _PAD), np.float32)
    pow_j = (_A ** np.arange(C, dtype=np.float64)).astype(np.float32)
    Tm[:, 0] = pow_j
    Tm[:, 1] = pow_j
    Tm[:, _PAD:] = _OMA * L
    Tv[:, _PAD:] = L
    POW = np.repeat(pow_j[:, None], 128, axis=1)
    w = (_OMA * _A ** np.arange(C - 1, -1, -1, dtype=np.float64))
    W = np.repeat(w.astype(np.float32)[:, None], 128, axis=1)
    return jnp.asarray(Tm), jnp.asarray(Tv), jnp.asarray(POW), jnp.asarray(W)


def _body(x_ref, m0_ref, v0_ref, tm_ref, tv_ref, pow_ref, w_ref,
          out_ref, mout_ref, vout_ref, rm_ref, rv_ref, mc_ref, vc_ref):
    b = pl.program_id(1)

    @pl.when(b == 0)
    def _init():
        rm_ref[0:_PAD, :] = jnp.zeros_like(rm_ref[0:_PAD, :])
        rv_ref[0:_PAD, :] = jnp.zeros_like(rv_ref[0:_PAD, :])
        m0 = m0_ref[...]
        hi = m0.astype(jnp.bfloat16).astype(jnp.float32)
        rm_ref[0:1, :] = hi
        rm_ref[1:2, :] = m0 - hi
        mc_ref[...] = m0
        vc_ref[...] = v0_ref[...]

    C = _C
    Fb = out_ref.shape[1]
    reps = Fb // 128
    xb = x_ref[...]
    rm_ref[_PAD:, :] = xb
    m = jax.lax.dot_general(
        tm_ref[...], rm_ref[...], (((1,), (0,)), ((), ())),
        precision=jax.lax.Precision.DEFAULT,
        preferred_element_type=jnp.float32)
    d = xb - m
    bb = (_A * _OMA) * (d * d)
    rv_ref[_PAD:, :] = bb
    powf = pltpu.repeat(pow_ref[...], reps, axis=1)
    vcarry = jnp.broadcast_to(vc_ref[...], (C, Fb))
    v = jax.lax.dot_general(
        tv_ref[...], rv_ref[...], (((1,), (0,)), ((), ())),
        precision=jax.lax.Precision.DEFAULT,
        preferred_element_type=jnp.float32) + powf * vcarry
    out_ref[...] = d * jax.lax.rsqrt(v + _EPS)

    wf = pltpu.repeat(w_ref[...], reps, axis=1)
    s = jnp.sum(wf * xb, axis=0, keepdims=True)
    m_carry = (_A ** C) * mc_ref[...] + s
    d_last = d[C - 1:C, :]
    v_carry = _A * v[C - 1:C, :] + (_A * _OMA) * (d_last * d_last)
    hi = m_carry.astype(jnp.bfloat16).astype(jnp.float32)
    rm_ref[0:1, :] = hi
    rm_ref[1:2, :] = m_carry - hi
    mc_ref[...] = m_carry
    vc_ref[...] = v_carry
    mout_ref[...] = m_carry
    vout_ref[...] = v_carry


def kernel(x, mstream, varstream):
    B, F = x.shape
    C = _C
    Fb = F // 2 if F % 256 == 0 and F >= 512 else F
    nb = B // C
    nf = F // Fb
    Tm, Tv, POW, W = _chunk_mats(C)
    m2 = mstream.reshape(1, F)
    v2 = varstream.reshape(1, F)

    out, mfin, vfin = pl.pallas_call(
        _body,
        grid=(nf, nb),
        in_specs=[
            pl.BlockSpec((C, Fb), lambda f, b: (b, f)),
            pl.BlockSpec((1, Fb), lambda f, b: (0, f)),
            pl.BlockSpec((1, Fb), lambda f, b: (0, f)),
            pl.BlockSpec((C, C + _PAD), lambda f, b: (0, 0)),
            pl.BlockSpec((C, C + _PAD), lambda f, b: (0, 0)),
            pl.BlockSpec((C, 128), lambda f, b: (0, 0)),
            pl.BlockSpec((C, 128), lambda f, b: (0, 0)),
        ],
        out_specs=[
            pl.BlockSpec((C, Fb), lambda f, b: (b, f)),
            pl.BlockSpec((1, Fb), lambda f, b: (0, f)),
            pl.BlockSpec((1, Fb), lambda f, b: (0, f)),
        ],
        out_shape=[
            jax.ShapeDtypeStruct((B, F), jnp.float32),
            jax.ShapeDtypeStruct((1, F), jnp.float32),
            jax.ShapeDtypeStruct((1, F), jnp.float32),
        ],
        scratch_shapes=[
            pltpu.VMEM((C + _PAD, Fb), jnp.float32),
            pltpu.VMEM((C + _PAD, Fb), jnp.float32),
            pltpu.VMEM((1, Fb), jnp.float32),
            pltpu.VMEM((1, Fb), jnp.float32),
        ],
        compiler_params=pltpu.CompilerParams(
            dimension_semantics=("parallel", "arbitrary")),
    )(x, m2, v2, Tm, Tv, POW, W)
    return out, mfin.reshape(F), vfin.reshape(F)


# speedup vs baseline: 1.3887x; 1.0020x over previous
import functools

import jax
import jax.numpy as jnp
import ml_dtypes
import numpy as np
from jax.experimental import pallas as pl
from jax.experimental.pallas import tpu as pltpu

_A = 0.999
_OMA = 1.0 - _A
_EPS = 1e-05
_C = 256
_PAD = 8


@functools.lru_cache(maxsize=None)
def _chunk_mats(C: int):
    j = np.arange(C, dtype=np.float64)[:, None]
    k = np.arange(C, dtype=np.float64)[None, :]
    L = np.where(k < j, _A ** np.maximum(j - 1 - k, 0.0), 0.0)
    Tm = np.zeros((C, C + _PAD), np.float32)
    Tv = np.zeros((C, C + _PAD), np.float32)
    pow_j = (_A ** np.arange(C, dtype=np.float64)).astype(np.float32)
    Tm[:, 0] = pow_j
    Tm[:, 1] = pow_j
    Tm[:, _PAD:] = _OMA * L
    pow_hi = pow_j.astype(ml_dtypes.bfloat16).astype(np.float32)
    pow_lo = pow_j - pow_hi
    Tv[:, 0] = pow_hi
    Tv[:, 1] = pow_hi
    Tv[:, 2] = pow_lo
    Tv[:, 3] = pow_lo
    Tv[:, _PAD:] = L
    w = (_OMA * _A ** np.arange(C - 1, -1, -1, dtype=np.float64))
    W = np.repeat(w.astype(np.float32)[:, None], 128, axis=1)
    return jnp.asarray(Tm), jnp.asarray(Tv), jnp.asarray(W)


def _store_vcarry(rv_ref, val):
    hi = val.astype(jnp.bfloat16).astype(jnp.float32)
    lo = val - hi
    rv_ref[0:1, :] = hi
    rv_ref[1:2, :] = lo
    rv_ref[2:3, :] = hi
    rv_ref[3:4, :] = lo


def _body(x_ref, m0_ref, v0_ref, tm_ref, tv_ref, w_ref,
          out_ref, mout_ref, vout_ref, rm_ref, rv_ref, mc_ref):
    b = pl.program_id(1)

    @pl.when(b == 0)
    def _init():
        rm_ref[0:_PAD, :] = jnp.zeros_like(rm_ref[0:_PAD, :])
        rv_ref[0:_PAD, :] = jnp.zeros_like(rv_ref[0:_PAD, :])
        m0 = m0_ref[...]
        hi = m0.astype(jnp.bfloat16).astype(jnp.float32)
        rm_ref[0:1, :] = hi
        rm_ref[1:2, :] = m0 - hi
        mc_ref[...] = m0
        _store_vcarry(rv_ref, v0_ref[...])

    C = _C
    Fb = out_ref.shape[1]
    reps = Fb // 128
    xb = x_ref[...]
    rm_ref[_PAD:, :] = xb
    m = jax.lax.dot_general(
        tm_ref[...], rm_ref[...], (((1,), (0,)), ((), ())),
        precision=jax.lax.Precision.DEFAULT,
        preferred_element_type=jnp.float32)
    d = xb - m
    bb = (_A * _OMA) * (d * d)
    rv_ref[_PAD:, :] = bb
    v = jax.lax.dot_general(
        tv_ref[...], rv_ref[...], (((1,), (0,)), ((), ())),
        precision=jax.lax.Precision.DEFAULT,
        preferred_element_type=jnp.float32)
    out_ref[...] = d * jax.lax.rsqrt(v + _EPS)

    wf = pltpu.repeat(w_ref[...], reps, axis=1)
    s = jnp.sum(wf * xb, axis=0, keepdims=True)
    m_carry = (_A ** C) * mc_ref[...] + s
    d_last = d[C - 1:C, :]
    v_carry = _A * v[C - 1:C, :] + (_A * _OMA) * (d_last * d_last)
    hi = m_carry.astype(jnp.bfloat16).astype(jnp.float32)
    rm_ref[0:1, :] = hi
    rm_ref[1:2, :] = m_carry - hi
    mc_ref[...] = m_carry
    _store_vcarry(rv_ref, v_carry)
    mout_ref[...] = m_carry
    vout_ref[...] = v_carry


def kernel(x, mstream, varstream):
    B, F = x.shape
    C = _C
    Fb = F // 2 if F % 256 == 0 and F >= 512 else F
    nb = B // C
    nf = F // Fb
    Tm, Tv, W = _chunk_mats(C)
    m2 = mstream.reshape(1, F)
    v2 = varstream.reshape(1, F)

    out, mfin, vfin = pl.pallas_call(
        _body,
        grid=(nf, nb),
        in_specs=[
            pl.BlockSpec((C, Fb), lambda f, b: (b, f)),
            pl.BlockSpec((1, Fb), lambda f, b: (0, f)),
            pl.BlockSpec((1, Fb), lambda f, b: (0, f)),
            pl.BlockSpec((C, C + _PAD), lambda f, b: (0, 0)),
            pl.BlockSpec((C, C + _PAD), lambda f, b: (0, 0)),
            pl.BlockSpec((C, 128), lambda f, b: (0, 0)),
        ],
        out_specs=[
            pl.BlockSpec((C, Fb), lambda f, b: (b, f)),
            pl.BlockSpec((1, Fb), lambda f, b: (0, f)),
            pl.BlockSpec((1, Fb), lambda f, b: (0, f)),
        ],
        out_shape=[
            jax.ShapeDtypeStruct((B, F), jnp.float32),
            jax.ShapeDtypeStruct((1, F), jnp.float32),
            jax.ShapeDtypeStruct((1, F), jnp.float32),
        ],
        scratch_shapes=[
            pltpu.VMEM((C + _PAD, Fb), jnp.float32),
            pltpu.VMEM((C + _PAD, Fb), jnp.float32),
            pltpu.VMEM((1, Fb), jnp.float32),
        ],
        compiler_params=pltpu.CompilerParams(
            dimension_semantics=("parallel", "arbitrary")),
    )(x, m2, v2, Tm, Tv, W)
    return out, mfin.reshape(F), vfin.reshape(F)


# speedup vs baseline: 1.4358x; 1.0339x over previous
import functools

import jax
import jax.numpy as jnp
import ml_dtypes
import numpy as np
from jax.experimental import pallas as pl
from jax.experimental.pallas import tpu as pltpu

_A = 0.999
_OMA = 1.0 - _A
_EPS = 1e-05
_C = 256
_PAD = 8


@functools.lru_cache(maxsize=None)
def _chunk_mats(C: int):
    j = np.arange(C, dtype=np.float64)[:, None]
    k = np.arange(C, dtype=np.float64)[None, :]
    L = np.where(k < j, _A ** np.maximum(j - 1 - k, 0.0), 0.0)
    Tm = np.zeros((C, C + _PAD), np.float32)
    Tv = np.zeros((C, C + _PAD), np.float32)
    pow_j = (_A ** np.arange(C, dtype=np.float64)).astype(np.float32)
    Tm[:, 0] = pow_j
    Tm[:, 1] = pow_j
    Tm[:, _PAD:] = _OMA * L
    pow_hi = pow_j.astype(ml_dtypes.bfloat16).astype(np.float32)
    pow_lo = pow_j - pow_hi
    Tv[:, 0] = pow_hi
    Tv[:, 1] = pow_hi
    Tv[:, 2] = pow_lo
    Tv[:, 3] = pow_lo
    Tv[:, _PAD:] = L
    w = (_OMA * _A ** np.arange(C - 1, -1, -1, dtype=np.float64))
    W = np.repeat(w.astype(np.float32)[:, None], 128, axis=1)
    return jnp.asarray(Tm), jnp.asarray(Tv), jnp.asarray(W)


def _store_vcarry(rv_ref, val):
    hi = val.astype(jnp.bfloat16).astype(jnp.float32)
    lo = val - hi
    rv_ref[0:1, :] = hi
    rv_ref[1:2, :] = lo
    rv_ref[2:3, :] = hi
    rv_ref[3:4, :] = lo


def _body(x_ref, m0_ref, v0_ref, tm_ref, tv_ref, w_ref,
          out_ref, mout_ref, vout_ref, rm_ref, rv_ref, mc_ref):
    b = pl.program_id(1)

    @pl.when(b == 0)
    def _init():
        rm_ref[0:_PAD, :] = jnp.zeros_like(rm_ref[0:_PAD, :])
        rv_ref[0:_PAD, :] = jnp.zeros_like(rv_ref[0:_PAD, :])
        m0 = m0_ref[...]
        hi = m0.astype(jnp.bfloat16).astype(jnp.float32)
        rm_ref[0:1, :] = hi
        rm_ref[1:2, :] = m0 - hi
        mc_ref[...] = m0
        _store_vcarry(rv_ref, v0_ref[...])

    C = _C
    Fb = out_ref.shape[1]
    reps = Fb // 128
    xb = x_ref[...]
    m = jax.lax.dot_general(
        tm_ref[...], jnp.concatenate([rm_ref[...], xb], axis=0),
        (((1,), (0,)), ((), ())),
        precision=jax.lax.Precision.DEFAULT,
        preferred_element_type=jnp.float32)
    d = xb - m
    bb = (_A * _OMA) * (d * d)
    v = jax.lax.dot_general(
        tv_ref[...], jnp.concatenate([rv_ref[...], bb], axis=0),
        (((1,), (0,)), ((), ())),
        precision=jax.lax.Precision.DEFAULT,
        preferred_element_type=jnp.float32)
    out_ref[...] = d * jax.lax.rsqrt(v + _EPS)

    wf = pltpu.repeat(w_ref[...], reps, axis=1)
    s = jnp.sum(wf * xb, axis=0, keepdims=True)
    m_carry = (_A ** C) * mc_ref[...] + s
    d_last = d[C - 1:C, :]
    v_carry = _A * v[C - 1:C, :] + (_A * _OMA) * (d_last * d_last)
    hi = m_carry.astype(jnp.bfloat16).astype(jnp.float32)
    rm_ref[0:1, :] = hi
    rm_ref[1:2, :] = m_carry - hi
    mc_ref[...] = m_carry
    _store_vcarry(rv_ref, v_carry)
    mout_ref[...] = m_carry
    vout_ref[...] = v_carry


def kernel(x, mstream, varstream):
    B, F = x.shape
    C = _C
    Fb = F // 2 if F % 256 == 0 and F >= 512 else F
    nb = B // C
    nf = F // Fb
    Tm, Tv, W = _chunk_mats(C)
    m2 = mstream.reshape(1, F)
    v2 = varstream.reshape(1, F)

    out, mfin, vfin = pl.pallas_call(
        _body,
        grid=(nf, nb),
        in_specs=[
            pl.BlockSpec((C, Fb), lambda f, b: (b, f)),
            pl.BlockSpec((1, Fb), lambda f, b: (0, f)),
            pl.BlockSpec((1, Fb), lambda f, b: (0, f)),
            pl.BlockSpec((C, C + _PAD), lambda f, b: (0, 0)),
            pl.BlockSpec((C, C + _PAD), lambda f, b: (0, 0)),
            pl.BlockSpec((C, 128), lambda f, b: (0, 0)),
        ],
        out_specs=[
            pl.BlockSpec((C, Fb), lambda f, b: (b, f)),
            pl.BlockSpec((1, Fb), lambda f, b: (0, f)),
            pl.BlockSpec((1, Fb), lambda f, b: (0, f)),
        ],
        out_shape=[
            jax.ShapeDtypeStruct((B, F), jnp.float32),
            jax.ShapeDtypeStruct((1, F), jnp.float32),
            jax.ShapeDtypeStruct((1, F), jnp.float32),
        ],
        scratch_shapes=[
            pltpu.VMEM((_PAD, Fb), jnp.float32),
            pltpu.VMEM((_PAD, Fb), jnp.float32),
            pltpu.VMEM((1, Fb), jnp.float32),
        ],
        compiler_params=pltpu.CompilerParams(
            dimension_semantics=("parallel", "arbitrary")),
    )(x, m2, v2, Tm, Tv, W)
    return out, mfin.reshape(F), vfin.reshape(F)
